# initial kernel scaffold (unmeasured)
import jax
import jax.numpy as jnp
from jax import lax
from jax.experimental import pallas as pl
from jax.experimental.pallas import tpu as pltpu

SCALE = 128 ** -0.5


def _flash_body(q_ref, k_ref, v_ref, acc_ref, m_ref, l_ref):
    q = q_ref[0].astype(jnp.bfloat16)
    k = k_ref[0].astype(jnp.bfloat16)
    v = v_ref[0].astype(jnp.bfloat16)

    s = lax.dot_general(
        q, k, (((2,), (2,)), ((1,), (1,))),
        preferred_element_type=jnp.float32,
    ) * SCALE
    m = jnp.max(s, axis=-1)
    p = jnp.exp(s - m[:, :, None])
    l = jnp.sum(p, axis=-1)
    pv = lax.dot_general(
        p.astype(jnp.bfloat16), v, (((2,), (0,)), ((0,), (1,))),
        preferred_element_type=jnp.float32,
    )
    acc_ref[0] = pv.transpose(1, 0, 2).astype(jnp.bfloat16)
    m_ref[0] = m
    l_ref[0] = l


def _combine_body(acc_ref, m_ref, l_ref, o_ref,
                  racc_ref, rm_ref, rl_ref, send_sems, recv_sems):
    my_x = lax.axis_index("x")
    my_y = lax.axis_index("y")
    my_z = lax.axis_index("z")
    peer = (1 - my_x, my_y, my_z)

    barrier = pltpu.get_barrier_semaphore()
    pl.semaphore_signal(barrier, inc=1, device_id=peer,
                        device_id_type=pl.DeviceIdType.MESH)
    pl.semaphore_wait(barrier, 1)

    copies = []
    for i, (src, dst) in enumerate(
        [(acc_ref, racc_ref), (m_ref, rm_ref), (l_ref, rl_ref)]
    ):
        c = pltpu.make_async_remote_copy(
            src_ref=src, dst_ref=dst,
            send_sem=send_sems.at[i], recv_sem=recv_sems.at[i],
            device_id=peer, device_id_type=pl.DeviceIdType.MESH,
        )
        c.start()
        copies.append(c)
    for c in copies:
        c.wait()

    m_a = m_ref[...]
    m_b = rm_ref[...]
    m_new = jnp.maximum(m_a, m_b)
    w_a = jnp.exp(m_a - m_new)
    w_b = jnp.exp(m_b - m_new)
    l_new = l_ref[...] * w_a + rl_ref[...] * w_b
    w_a = w_a.transpose(0, 2, 1)[..., None]
    w_b = w_b.transpose(0, 2, 1)[..., None]
    l_new = l_new.transpose(0, 2, 1)[..., None]
    acc = (acc_ref[...].astype(jnp.float32) * w_a
           + racc_ref[...].astype(jnp.float32) * w_b)
    o_ref[...] = acc / l_new


def kernel(Q, K, V):
    b, sq, h, d = Q.shape
    skv = K.shape[1]

    acc, m, l = pl.pallas_call(
        _flash_body,
        grid=(b,),
        in_specs=[
            pl.BlockSpec((1, sq, h, d), lambda i: (i, 0, 0, 0)),
            pl.BlockSpec((1, skv, h, d), lambda i: (i, 0, 0, 0)),
            pl.BlockSpec((1, skv, h, d), lambda i: (i, 0, 0, 0)),
        ],
        out_specs=[
            pl.BlockSpec((1, sq, h, d), lambda i: (i, 0, 0, 0)),
            pl.BlockSpec((1, h, sq), lambda i: (i, 0, 0)),
            pl.BlockSpec((1, h, sq), lambda i: (i, 0, 0)),
        ],
        out_shape=[
            jax.ShapeDtypeStruct((b, sq, h, d), jnp.bfloat16),
            jax.ShapeDtypeStruct((b, h, sq), jnp.float32),
            jax.ShapeDtypeStruct((b, h, sq), jnp.float32),
        ],
    )(Q, K, V)

    return pl.pallas_call(
        _combine_body,
        out_shape=jax.ShapeDtypeStruct((b, sq, h, d), jnp.float32),
        in_specs=[pl.BlockSpec(memory_space=pltpu.VMEM)] * 3,
        out_specs=pl.BlockSpec(memory_space=pltpu.VMEM),
        scratch_shapes=[
            pltpu.VMEM((b, sq, h, d), jnp.bfloat16),
            pltpu.VMEM((b, h, sq), jnp.float32),
            pltpu.VMEM((b, h, sq), jnp.float32),
            pltpu.SemaphoreType.DMA((3,)),
            pltpu.SemaphoreType.DMA((3,)),
        ],
        compiler_params=pltpu.CompilerParams(collective_id=0),
    )(acc, m, l)


# baseline (device time: 156909 ns/iter reference)
import jax
import jax.numpy as jnp
from jax import lax
from jax.experimental import pallas as pl
from jax.experimental.pallas import tpu as pltpu

SCALE = 128 ** -0.5


H_BLK = 8


def _flash_body(q_ref, k_ref, v_ref, acc_ref, m_ref, l_ref):
    ms, ls = [], []
    for jh in range(H_BLK):
        q = q_ref[0, :, jh, :].astype(jnp.bfloat16)
        k = k_ref[0, :, jh, :].astype(jnp.bfloat16)
        v = v_ref[0, :, jh, :].astype(jnp.bfloat16)

        s = lax.dot_general(
            q, k, (((1,), (1,)), ((), ())),
            preferred_element_type=jnp.float32,
        ) * SCALE
        m = jnp.max(s, axis=-1)
        p = jnp.exp(s - m[:, None])
        l = jnp.sum(p, axis=-1)
        pv = lax.dot_general(
            p.astype(jnp.bfloat16), v, (((1,), (0,)), ((), ())),
            preferred_element_type=jnp.float32,
        )
        acc_ref[0, :, jh, :] = pv.astype(jnp.bfloat16)
        ms.append(m)
        ls.append(l)
    m_ref[0] = jnp.stack(ms)
    l_ref[0] = jnp.stack(ls)


def _combine_body(acc_ref, m_ref, l_ref, o_ref,
                  racc_ref, rm_ref, rl_ref, send_sems, recv_sems):
    my_x = lax.axis_index("x")
    my_y = lax.axis_index("y")
    my_z = lax.axis_index("z")
    peer = (1 - my_x, my_y, my_z)

    barrier = pltpu.get_barrier_semaphore()
    pl.semaphore_signal(barrier, inc=1, device_id=peer,
                        device_id_type=pl.DeviceIdType.MESH)
    pl.semaphore_wait(barrier, 1)

    copies = []
    for i, (src, dst) in enumerate(
        [(acc_ref, racc_ref), (m_ref, rm_ref), (l_ref, rl_ref)]
    ):
        c = pltpu.make_async_remote_copy(
            src_ref=src, dst_ref=dst,
            send_sem=send_sems.at[i], recv_sem=recv_sems.at[i],
            device_id=peer, device_id_type=pl.DeviceIdType.MESH,
        )
        c.start()
        copies.append(c)
    for c in copies:
        c.wait()

    m_a = m_ref[...]
    m_b = rm_ref[...]
    m_new = jnp.maximum(m_a, m_b)
    w_a = jnp.exp(m_a - m_new)
    w_b = jnp.exp(m_b - m_new)
    l_new = l_ref[...] * w_a + rl_ref[...] * w_b
    w_a = w_a.transpose(0, 2, 1)[..., None]
    w_b = w_b.transpose(0, 2, 1)[..., None]
    l_new = l_new.transpose(0, 2, 1)[..., None]
    acc = (acc_ref[...].astype(jnp.float32) * w_a
           + racc_ref[...].astype(jnp.float32) * w_b)
    o_ref[...] = acc / l_new


def kernel(Q, K, V):
    b, sq, h, d = Q.shape
    skv = K.shape[1]

    acc, m, l = pl.pallas_call(
        _flash_body,
        grid=(b, h // H_BLK),
        in_specs=[
            pl.BlockSpec((1, sq, H_BLK, d), lambda i, j: (i, 0, j, 0)),
            pl.BlockSpec((1, skv, H_BLK, d), lambda i, j: (i, 0, j, 0)),
            pl.BlockSpec((1, skv, H_BLK, d), lambda i, j: (i, 0, j, 0)),
        ],
        out_specs=[
            pl.BlockSpec((1, sq, H_BLK, d), lambda i, j: (i, 0, j, 0)),
            pl.BlockSpec((1, H_BLK, sq), lambda i, j: (i, j, 0)),
            pl.BlockSpec((1, H_BLK, sq), lambda i, j: (i, j, 0)),
        ],
        out_shape=[
            jax.ShapeDtypeStruct((b, sq, h, d), jnp.bfloat16),
            jax.ShapeDtypeStruct((b, h, sq), jnp.float32),
            jax.ShapeDtypeStruct((b, h, sq), jnp.float32),
        ],
    )(Q, K, V)

    return pl.pallas_call(
        _combine_body,
        out_shape=jax.ShapeDtypeStruct((b, sq, h, d), jnp.float32),
        in_specs=[pl.BlockSpec(memory_space=pltpu.VMEM)] * 3,
        out_specs=pl.BlockSpec(memory_space=pltpu.VMEM),
        scratch_shapes=[
            pltpu.VMEM((b, sq, h, d), jnp.bfloat16),
            pltpu.VMEM((b, h, sq), jnp.float32),
            pltpu.VMEM((b, h, sq), jnp.float32),
            pltpu.SemaphoreType.DMA((3,)),
            pltpu.SemaphoreType.DMA((3,)),
        ],
        compiler_params=pltpu.CompilerParams(collective_id=0),
    )(acc, m, l)


# device time: 93569 ns/iter; 1.6769x vs baseline; 1.6769x over previous
import jax
import jax.numpy as jnp
from jax import lax
from jax.experimental import pallas as pl
from jax.experimental.pallas import tpu as pltpu

SCALE = 128 ** -0.5


H_BLK = 8


def _flash_body(q_ref, k_ref, v_ref, acc_ref, m_ref, l_ref):
    ms, ls = [], []
    for jh in range(H_BLK):
        q = q_ref[0, :, jh, :]
        k = k_ref[0, :, jh, :]
        v = v_ref[0, :, jh, :]

        s = lax.dot_general(
            q, k, (((1,), (1,)), ((), ())),
            preferred_element_type=jnp.float32,
        ) * SCALE
        m = jnp.max(s, axis=-1)
        p = jnp.exp(s - m[:, None])
        l = jnp.sum(p, axis=-1)
        pv = lax.dot_general(
            p, v, (((1,), (0,)), ((), ())),
            preferred_element_type=jnp.float32,
        )
        acc_ref[0, :, jh, :] = pv.astype(jnp.bfloat16)
        ms.append(m)
        ls.append(l)
    m_ref[0] = jnp.stack(ms)
    l_ref[0] = jnp.stack(ls)


def _combine_body(acc_ref, m_ref, l_ref, o_ref,
                  racc_ref, rm_ref, rl_ref, send_sems, recv_sems):
    my_x = lax.axis_index("x")
    my_y = lax.axis_index("y")
    my_z = lax.axis_index("z")
    peer = (1 - my_x, my_y, my_z)

    barrier = pltpu.get_barrier_semaphore()
    pl.semaphore_signal(barrier, inc=1, device_id=peer,
                        device_id_type=pl.DeviceIdType.MESH)
    pl.semaphore_wait(barrier, 1)

    copies = []
    for i, (src, dst) in enumerate(
        [(acc_ref, racc_ref), (m_ref, rm_ref), (l_ref, rl_ref)]
    ):
        c = pltpu.make_async_remote_copy(
            src_ref=src, dst_ref=dst,
            send_sem=send_sems.at[i], recv_sem=recv_sems.at[i],
            device_id=peer, device_id_type=pl.DeviceIdType.MESH,
        )
        c.start()
        copies.append(c)
    for c in copies:
        c.wait()

    m_a = m_ref[...]
    m_b = rm_ref[...]
    m_new = jnp.maximum(m_a, m_b)
    w_a = jnp.exp(m_a - m_new)
    w_b = jnp.exp(m_b - m_new)
    l_new = l_ref[...] * w_a + rl_ref[...] * w_b
    w_a = w_a.transpose(0, 2, 1)[..., None]
    w_b = w_b.transpose(0, 2, 1)[..., None]
    l_new = l_new.transpose(0, 2, 1)[..., None]
    acc = (acc_ref[...].astype(jnp.float32) * w_a
           + racc_ref[...].astype(jnp.float32) * w_b)
    o_ref[...] = acc / l_new


def kernel(Q, K, V):
    b, sq, h, d = Q.shape
    skv = K.shape[1]

    acc, m, l = pl.pallas_call(
        _flash_body,
        grid=(b, h // H_BLK),
        in_specs=[
            pl.BlockSpec((1, sq, H_BLK, d), lambda i, j: (i, 0, j, 0)),
            pl.BlockSpec((1, skv, H_BLK, d), lambda i, j: (i, 0, j, 0)),
            pl.BlockSpec((1, skv, H_BLK, d), lambda i, j: (i, 0, j, 0)),
        ],
        out_specs=[
            pl.BlockSpec((1, sq, H_BLK, d), lambda i, j: (i, 0, j, 0)),
            pl.BlockSpec((1, H_BLK, sq), lambda i, j: (i, j, 0)),
            pl.BlockSpec((1, H_BLK, sq), lambda i, j: (i, j, 0)),
        ],
        out_shape=[
            jax.ShapeDtypeStruct((b, sq, h, d), jnp.bfloat16),
            jax.ShapeDtypeStruct((b, h, sq), jnp.float32),
            jax.ShapeDtypeStruct((b, h, sq), jnp.float32),
        ],
    )(Q, K, V)

    return pl.pallas_call(
        _combine_body,
        out_shape=jax.ShapeDtypeStruct((b, sq, h, d), jnp.float32),
        in_specs=[pl.BlockSpec(memory_space=pltpu.VMEM)] * 3,
        out_specs=pl.BlockSpec(memory_space=pltpu.VMEM),
        scratch_shapes=[
            pltpu.VMEM((b, sq, h, d), jnp.bfloat16),
            pltpu.VMEM((b, h, sq), jnp.float32),
            pltpu.VMEM((b, h, sq), jnp.float32),
            pltpu.SemaphoreType.DMA((3,)),
            pltpu.SemaphoreType.DMA((3,)),
        ],
        compiler_params=pltpu.CompilerParams(collective_id=0),
    )(acc, m, l)


# device time: 52122 ns/iter; 3.0104x vs baseline; 1.7952x over previous
import jax
import jax.numpy as jnp
from jax import lax
from jax.experimental import pallas as pl
from jax.experimental.pallas import tpu as pltpu

SCALE = 128 ** -0.5
SCALE2 = SCALE * 1.4426950408889634
H_BLK = 8


def _fused_body(q_ref, k_ref, v_ref, o_ref,
                acc_s, ml_s, racc_s, rml_s,
                send_sems, recv_sems):
    i = pl.program_id(0)
    j = pl.program_id(1)

    my_x = lax.axis_index("x")
    my_y = lax.axis_index("y")
    my_z = lax.axis_index("z")
    peer = (1 - my_x, my_y, my_z)
    barrier = pltpu.get_barrier_semaphore()

    @pl.when((i == 0) & (j == 0))
    def _():
        pl.semaphore_signal(barrier, inc=1, device_id=peer,
                            device_id_type=pl.DeviceIdType.MESH)
        pl.semaphore_wait(barrier, 1)

    kk = k_ref[0].astype(jnp.bfloat16).transpose(1, 0, 2)
    vv = v_ref[0].astype(jnp.bfloat16).transpose(1, 0, 2)
    qq = q_ref[0].astype(jnp.bfloat16).transpose(1, 0, 2)

    s = lax.dot_general(
        qq, kk, (((2,), (2,)), ((0,), (0,))),
        preferred_element_type=jnp.float32,
    ) * SCALE2
    m = jnp.max(s, axis=-1)
    p = jnp.exp2(s - m[:, :, None])
    l = jnp.sum(p, axis=-1)
    pv = lax.dot_general(
        p.astype(jnp.bfloat16), vv, (((2,), (1,)), ((0,), (0,))),
        preferred_element_type=jnp.float32,
    )
    pv_t = pv.transpose(1, 0, 2).astype(jnp.bfloat16)

    @pl.when(j == 0)
    def _():
        acc_s[i, :, 0:H_BLK, :] = pv_t
        ml_s[i, 0, 0:H_BLK, :] = m
        ml_s[i, 1, 0:H_BLK, :] = l

    def _rdmas(c):
        acc_rdma = pltpu.make_async_remote_copy(
            src_ref=acc_s.at[c], dst_ref=racc_s.at[c],
            send_sem=send_sems.at[0, c], recv_sem=recv_sems.at[0, c],
            device_id=peer, device_id_type=pl.DeviceIdType.MESH,
        )
        ml_rdma = pltpu.make_async_remote_copy(
            src_ref=ml_s.at[c], dst_ref=rml_s.at[c],
            send_sem=send_sems.at[1, c], recv_sem=recv_sems.at[1, c],
            device_id=peer, device_id_type=pl.DeviceIdType.MESH,
        )
        return acc_rdma, ml_rdma

    def _combine(c):
        acc_rdma, ml_rdma = _rdmas(c)
        ml_rdma.wait()
        acc_rdma.wait()
        m_a = ml_s[c, 0]
        m_b = rml_s[c, 0]
        m_new = jnp.maximum(m_a, m_b)
        w_a = jnp.exp2(m_a - m_new)
        w_b = jnp.exp2(m_b - m_new)
        l_new = ml_s[c, 1] * w_a + rml_s[c, 1] * w_b
        w_a = w_a.transpose(1, 0)[..., None]
        w_b = w_b.transpose(1, 0)[..., None]
        l_new = l_new.transpose(1, 0)[..., None]
        acc = (acc_s[c].astype(jnp.float32) * w_a
               + racc_s[c].astype(jnp.float32) * w_b)
        o_ref[0] = acc / l_new

    @pl.when((j == 0) & (i > 0))
    def _():
        _combine(i - 1)

    @pl.when(j == 1)
    def _():
        acc_s[i, :, H_BLK:2 * H_BLK, :] = pv_t
        ml_s[i, 0, H_BLK:2 * H_BLK, :] = m
        ml_s[i, 1, H_BLK:2 * H_BLK, :] = l

        acc_rdma, ml_rdma = _rdmas(i)
        acc_rdma.start()
        ml_rdma.start()

    @pl.when((j == 1) & (i == pl.num_programs(0) - 1))
    def _():
        _combine(i)


def kernel(Q, K, V):
    b, sq, h, d = Q.shape
    skv = K.shape[1]

    return pl.pallas_call(
        _fused_body,
        grid=(b, h // H_BLK),
        in_specs=[
            pl.BlockSpec((1, sq, H_BLK, d), lambda i, j: (i, 0, j, 0)),
            pl.BlockSpec((1, skv, H_BLK, d), lambda i, j: (i, 0, j, 0)),
            pl.BlockSpec((1, skv, H_BLK, d), lambda i, j: (i, 0, j, 0)),
        ],
        out_specs=pl.BlockSpec(
            (1, sq, h, d), lambda i, j: ((i - 1 + j) % b, 0, 0, 0)
        ),
        out_shape=jax.ShapeDtypeStruct((b, sq, h, d), jnp.float32),
        scratch_shapes=[
            pltpu.VMEM((b, sq, h, d), jnp.bfloat16),
            pltpu.VMEM((b, 2, h, sq), jnp.float32),
            pltpu.VMEM((b, sq, h, d), jnp.bfloat16),
            pltpu.VMEM((b, 2, h, sq), jnp.float32),
            pltpu.SemaphoreType.DMA((2, b)),
            pltpu.SemaphoreType.DMA((2, b)),
        ],
        compiler_params=pltpu.CompilerParams(collective_id=0),
    )(Q, K, V)
